# baseline (speedup 1.0000x reference)
import functools

import jax
import jax.numpy as jnp
from jax import lax
from jax.experimental import pallas as pl
from jax.experimental.pallas import tpu as pltpu
from jax.experimental.pallas import tpu_sc as plsc

N = 10000
E = 320000
D_IN = 128
D_HID = 128
D_OUT = 64
NPAD = 10240
NROW = 80

_NSC = 2
_NTILE = 16
_EDGES_PER_SC = E // _NSC
_EDGES_PER_TILE = _EDGES_PER_SC // _NTILE
_K = 80
_NCHUNK = _EDGES_PER_TILE // _K
_ROWS_PER_TILE = NPAD // _NTILE

_BR = 128
_NBLK = (N + _BR - 1) // _BR


@functools.lru_cache(maxsize=None)
def _make_segsum():
    mesh = plsc.VectorSubcoreMesh(core_axis_name="c", subcore_axis_name="s")

    @functools.partial(
        pl.kernel,
        out_type=jax.ShapeDtypeStruct((_NSC, NPAD, 128), jnp.float32),
        mesh=mesh,
        scratch_types=[
            pltpu.VMEM((_K,), jnp.int32),
            pltpu.VMEM((_K,), jnp.int32),
            pltpu.VMEM((_K, 128), jnp.float32),
            pltpu.VMEM((_K,), jnp.int32),
            pltpu.VMEM((_K,), jnp.int32),
            pltpu.VMEM((_K, 128), jnp.float32),
            pltpu.VMEM((_K,), jnp.int32),
            pltpu.VMEM((_K,), jnp.int32),
            pltpu.VMEM((_K, 128), jnp.float32),
            pltpu.VMEM_SHARED((NPAD, 128), jnp.float32),
            pltpu.SemaphoreType.DMA,
            pltpu.SemaphoreType.DMA,
            pltpu.SemaphoreType.DMA,
        ],
    )
    def seg(p_hbm, src_hbm, dst_hbm, zero_hbm, out_hbm,
            src_a, dst_a, rows_a, src_b, dst_b, rows_b,
            src_c, dst_c, rows_c, acc_sh, sem_a, sem_b, sem_c):
        c = lax.axis_index("c")
        s = lax.axis_index("s")
        r0 = s * _ROWS_PER_TILE

        pltpu.sync_copy(zero_hbm.at[pl.ds(r0, _ROWS_PER_TILE)],
                        acc_sh.at[pl.ds(r0, _ROWS_PER_TILE)])

        base0 = c * _EDGES_PER_SC + s * _EDGES_PER_TILE

        def idx_load(i, sv, dv):
            b = base0 + i * _K
            pltpu.sync_copy(src_hbm.at[pl.ds(b, _K)], sv)
            pltpu.sync_copy(dst_hbm.at[pl.ds(b, _K)], dv)

        idx_load(0, src_a, dst_a)
        pltpu.async_copy(p_hbm.at[src_a], rows_a, sem_a)
        idx_load(1, src_b, dst_b)
        pltpu.async_copy(p_hbm.at[src_b], rows_b, sem_b)
        plsc.subcore_barrier()

        def body(t, carry):
            idx_load(3 * t + 2, src_c, dst_c)
            pltpu.async_copy(p_hbm.at[src_c], rows_c, sem_c)
            pltpu.make_async_copy(p_hbm.at[src_a], rows_a, sem_a).wait()
            pltpu.sync_copy(rows_a, acc_sh.at[dst_a], add=True)
            idx_load(3 * t + 3, src_a, dst_a)
            pltpu.async_copy(p_hbm.at[src_a], rows_a, sem_a)
            pltpu.make_async_copy(p_hbm.at[src_b], rows_b, sem_b).wait()
            pltpu.sync_copy(rows_b, acc_sh.at[dst_b], add=True)
            idx_load(3 * t + 4, src_b, dst_b)
            pltpu.async_copy(p_hbm.at[src_b], rows_b, sem_b)
            pltpu.make_async_copy(p_hbm.at[src_c], rows_c, sem_c).wait()
            pltpu.sync_copy(rows_c, acc_sh.at[dst_c], add=True)
            return carry

        lax.fori_loop(0, (_NCHUNK - 2) // 3, body, 0)
        pltpu.make_async_copy(p_hbm.at[src_a], rows_a, sem_a).wait()
        pltpu.sync_copy(rows_a, acc_sh.at[dst_a], add=True)
        pltpu.make_async_copy(p_hbm.at[src_b], rows_b, sem_b).wait()
        pltpu.sync_copy(rows_b, acc_sh.at[dst_b], add=True)
        plsc.subcore_barrier()
        pltpu.sync_copy(acc_sh.at[pl.ds(r0, _ROWS_PER_TILE)],
                        out_hbm.at[c, pl.ds(r0, _ROWS_PER_TILE)])

    return seg


_HCH = 2000
_NHBLK = E // _HCH


def _hist_body(d_ref, out_ref):
    i = pl.program_id(0)
    d = d_ref[...]
    lane = lax.broadcasted_iota(jnp.int32, (1, 128), 1)
    ohh = (lax.shift_right_logical(d, 7) == lane).astype(jnp.float32)
    ohl = (lax.bitwise_and(d, 127) == lane).astype(jnp.float32)
    g = lax.dot_general(ohh, ohl, (((0,), (0,)), ((), ())),
                        preferred_element_type=jnp.float32)

    @pl.when(i == 0)
    def _():
        out_ref[...] = g

    @pl.when(i > 0)
    def _():
        out_ref[...] += g


def _pre_body(x_ref, wl_ref, out_ref):
    out_ref[...] = jnp.dot(x_ref[...], wl_ref[...],
                           preferred_element_type=jnp.float32)


def _inv_diag(cp_ref):
    cnt = cp_ref[0, 0]
    inv = 1.0 / jnp.maximum(cnt, 1.0)
    return jnp.eye(_BR, dtype=jnp.float32) * inv[None, :]


def _mid_body(sp_ref, cp_ref, x_ref, wr_ref, b1_ref, w2l_ref, h_ref, p2_ref):
    ssum = sp_ref[0] + sp_ref[1]
    mean_w = jnp.dot(_inv_diag(cp_ref), ssum,
                     preferred_element_type=jnp.float32)
    h = mean_w + jnp.dot(x_ref[...], wr_ref[...],
                         preferred_element_type=jnp.float32) + b1_ref[...]
    h = jnp.maximum(h, 0.0)
    h_ref[...] = h
    p2 = jnp.dot(h, w2l_ref[...], preferred_element_type=jnp.float32)
    p2_ref[...] = jnp.concatenate(
        [p2, jnp.zeros((_BR, 128 - D_OUT), jnp.float32)], axis=1)


def _out_body(sp_ref, cp_ref, h_ref, w2r_ref, b2_ref, out_ref):
    ssum = sp_ref[0, :, :D_OUT] + sp_ref[1, :, :D_OUT]
    t = (jnp.dot(_inv_diag(cp_ref), ssum, preferred_element_type=jnp.float32)
         + jnp.dot(h_ref[...], w2r_ref[...], preferred_element_type=jnp.float32)
         + b2_ref[...])
    m = jnp.max(t, axis=1, keepdims=True)
    lse = m + jnp.log(jnp.sum(jnp.exp(t - m), axis=1, keepdims=True))
    out_ref[...] = t - lse


def _segsum(p, src, dst, zeros):
    return _make_segsum()(p, src, dst, zeros)


def kernel(x, edge_index, W1_l, W1_r, b1, W2_l, W2_r, b2):
    src = edge_index[0]
    dst = edge_index[1]

    b1r = b1.reshape(1, D_HID)
    b2r = b2.reshape(1, D_OUT)
    zeros = jnp.zeros((NPAD, 128), jnp.float32)

    p1 = pl.pallas_call(
        _pre_body,
        grid=(_NBLK,),
        in_specs=[pl.BlockSpec((_BR, D_IN), lambda i: (i, 0)),
                  pl.BlockSpec((D_IN, D_HID), lambda i: (0, 0))],
        out_specs=pl.BlockSpec((_BR, D_HID), lambda i: (i, 0)),
        out_shape=jax.ShapeDtypeStruct((N, D_HID), jnp.float32),
    )(x, W1_l)

    cntg = pl.pallas_call(
        _hist_body,
        grid=(_NHBLK,),
        in_specs=[pl.BlockSpec((_HCH, 1), lambda i: (i, 0))],
        out_specs=pl.BlockSpec((128, 128), lambda i: (0, 0)),
        out_shape=jax.ShapeDtypeStruct((128, 128), jnp.float32),
    )(dst.reshape(E, 1))
    cnt4 = cntg.reshape(128, 1, 128)

    s1p = _segsum(p1, src, dst, zeros)

    h, p2 = pl.pallas_call(
        _mid_body,
        grid=(_NBLK,),
        in_specs=[pl.BlockSpec((_NSC, _BR, 128), lambda i: (0, i, 0)),
                  pl.BlockSpec((1, 1, 128), lambda i: (i, 0, 0)),
                  pl.BlockSpec((_BR, D_IN), lambda i: (i, 0)),
                  pl.BlockSpec((D_IN, D_HID), lambda i: (0, 0)),
                  pl.BlockSpec((1, D_HID), lambda i: (0, 0)),
                  pl.BlockSpec((D_HID, D_OUT), lambda i: (0, 0))],
        out_specs=[pl.BlockSpec((_BR, D_HID), lambda i: (i, 0)),
                   pl.BlockSpec((_BR, 128), lambda i: (i, 0))],
        out_shape=[jax.ShapeDtypeStruct((N, D_HID), jnp.float32),
                   jax.ShapeDtypeStruct((N, 128), jnp.float32)],
    )(s1p, cnt4, x, W1_r, b1r, W2_l)

    s2p = _segsum(p2, src, dst, zeros)

    out = pl.pallas_call(
        _out_body,
        grid=(_NBLK,),
        in_specs=[pl.BlockSpec((_NSC, _BR, 128), lambda i: (0, i, 0)),
                  pl.BlockSpec((1, 1, 128), lambda i: (i, 0, 0)),
                  pl.BlockSpec((_BR, D_HID), lambda i: (i, 0)),
                  pl.BlockSpec((D_HID, D_OUT), lambda i: (0, 0)),
                  pl.BlockSpec((1, D_OUT), lambda i: (0, 0))],
        out_specs=pl.BlockSpec((_BR, D_OUT), lambda i: (i, 0)),
        out_shape=jax.ShapeDtypeStruct((N, D_OUT), jnp.float32),
    )(s2p, cnt4, h, W2_r, b2r)
    return out


# speedup vs baseline: 1.0588x; 1.0588x over previous
import functools

import jax
import jax.numpy as jnp
from jax import lax
from jax.experimental import pallas as pl
from jax.experimental.pallas import tpu as pltpu
from jax.experimental.pallas import tpu_sc as plsc

N = 10000
E = 320000
D_IN = 128
D_HID = 128
D_OUT = 64
NPAD = 10240
NROW = 80

_NSC = 2
_NTILE = 16
_EDGES_PER_SC = E // _NSC
_EDGES_PER_TILE = _EDGES_PER_SC // _NTILE
_K = 80
_NCHUNK = _EDGES_PER_TILE // _K
_ROWS_PER_TILE = NPAD // _NTILE

_BR = 128
_NBLK = (N + _BR - 1) // _BR


@functools.lru_cache(maxsize=None)
def _make_segsum():
    mesh = plsc.VectorSubcoreMesh(core_axis_name="c", subcore_axis_name="s")

    @functools.partial(
        pl.kernel,
        out_type=jax.ShapeDtypeStruct((_NSC, NPAD, 128), jnp.float32),
        mesh=mesh,
        scratch_types=[
            pltpu.VMEM((_K,), jnp.int32),
            pltpu.VMEM((_K,), jnp.int32),
            pltpu.VMEM((_K, 128), jnp.float32),
            pltpu.VMEM((_K,), jnp.int32),
            pltpu.VMEM((_K,), jnp.int32),
            pltpu.VMEM((_K, 128), jnp.float32),
            pltpu.VMEM((_K,), jnp.int32),
            pltpu.VMEM((_K,), jnp.int32),
            pltpu.VMEM((_K, 128), jnp.float32),
            pltpu.VMEM_SHARED((NPAD, 128), jnp.float32),
        ] + [pltpu.SemaphoreType.DMA] * 6,
    )
    def seg(p_hbm, src_hbm, dst_hbm, zero_hbm, out_hbm,
            src_a, dst_a, rows_a, src_b, dst_b, rows_b,
            src_c, dst_c, rows_c, acc_sh,
            sem_ia, sem_ib, sem_ic, sem_ga, sem_gb, sem_gc):
        c = lax.axis_index("c")
        s = lax.axis_index("s")
        r0 = s * _ROWS_PER_TILE

        pltpu.sync_copy(zero_hbm.at[pl.ds(r0, _ROWS_PER_TILE)],
                        acc_sh.at[pl.ds(r0, _ROWS_PER_TILE)])

        base0 = c * _EDGES_PER_SC + s * _EDGES_PER_TILE
        srcs = (src_a, src_b, src_c)
        dsts = (dst_a, dst_b, dst_c)
        rows = (rows_a, rows_b, rows_c)
        sem_i = (sem_ia, sem_ib, sem_ic)
        sem_g = (sem_ga, sem_gb, sem_gc)

        def idx_fire(i, k):
            b = base0 + i * _K
            pltpu.async_copy(src_hbm.at[pl.ds(b, _K)], srcs[k], sem_i[k])
            pltpu.async_copy(dst_hbm.at[pl.ds(b, _K)], dsts[k], sem_i[k])

        def idx_wait(i, k):
            b = base0 + i * _K
            pltpu.make_async_copy(src_hbm.at[pl.ds(b, _K)], srcs[k],
                                  sem_i[k]).wait()
            pltpu.make_async_copy(dst_hbm.at[pl.ds(b, _K)], dsts[k],
                                  sem_i[k]).wait()

        def gather_fire(k):
            pltpu.async_copy(p_hbm.at[srcs[k]], rows[k], sem_g[k])

        def gather_wait(k):
            pltpu.make_async_copy(p_hbm.at[srcs[k]], rows[k],
                                  sem_g[k]).wait()

        def scatter(k):
            pltpu.sync_copy(rows[k], acc_sh.at[dsts[k]], add=True)

        idx_fire(0, 0)
        idx_fire(1, 1)
        idx_wait(0, 0)
        gather_fire(0)
        idx_wait(1, 1)
        gather_fire(1)
        idx_fire(2, 2)
        plsc.subcore_barrier()

        def body(t, carry):
            for u in range(3):
                g = 3 * t + u
                this, prev = u, (u + 2) % 3
                idx_wait(g + 2, prev)
                gather_fire(prev)
                gather_wait(this)
                scatter(this)

                @pl.when(g + 3 < _NCHUNK)
                def _(g=g, this=this):
                    idx_fire(g + 3, this)
            return carry

        lax.fori_loop(0, (_NCHUNK - 2) // 3, body, 0)
        gather_wait(0)
        scatter(0)
        gather_wait(1)
        scatter(1)
        plsc.subcore_barrier()
        pltpu.sync_copy(acc_sh.at[pl.ds(r0, _ROWS_PER_TILE)],
                        out_hbm.at[c, pl.ds(r0, _ROWS_PER_TILE)])

    return seg


_HCH = 2000
_NHBLK = E // _HCH


def _hist_body(d_ref, out_ref):
    i = pl.program_id(0)
    d = d_ref[...]
    lane = lax.broadcasted_iota(jnp.int32, (1, 128), 1)
    ohh = (lax.shift_right_logical(d, 7) == lane).astype(jnp.float32)
    ohl = (lax.bitwise_and(d, 127) == lane).astype(jnp.float32)
    g = lax.dot_general(ohh, ohl, (((0,), (0,)), ((), ())),
                        preferred_element_type=jnp.float32)

    @pl.when(i == 0)
    def _():
        out_ref[...] = g

    @pl.when(i > 0)
    def _():
        out_ref[...] += g


def _pre_body(x_ref, wl_ref, out_ref):
    out_ref[...] = jnp.dot(x_ref[...], wl_ref[...],
                           preferred_element_type=jnp.float32)


def _inv_diag(cp_ref):
    cnt = cp_ref[0, 0]
    inv = 1.0 / jnp.maximum(cnt, 1.0)
    return jnp.eye(_BR, dtype=jnp.float32) * inv[None, :]


def _mid_body(sp_ref, cp_ref, x_ref, wr_ref, b1_ref, w2l_ref, h_ref, p2_ref):
    ssum = sp_ref[0] + sp_ref[1]
    mean_w = jnp.dot(_inv_diag(cp_ref), ssum,
                     preferred_element_type=jnp.float32)
    h = mean_w + jnp.dot(x_ref[...], wr_ref[...],
                         preferred_element_type=jnp.float32) + b1_ref[...]
    h = jnp.maximum(h, 0.0)
    h_ref[...] = h
    p2 = jnp.dot(h, w2l_ref[...], preferred_element_type=jnp.float32)
    p2_ref[...] = jnp.concatenate(
        [p2, jnp.zeros((_BR, 128 - D_OUT), jnp.float32)], axis=1)


def _out_body(sp_ref, cp_ref, h_ref, w2r_ref, b2_ref, out_ref):
    ssum = sp_ref[0, :, :D_OUT] + sp_ref[1, :, :D_OUT]
    t = (jnp.dot(_inv_diag(cp_ref), ssum, preferred_element_type=jnp.float32)
         + jnp.dot(h_ref[...], w2r_ref[...], preferred_element_type=jnp.float32)
         + b2_ref[...])
    m = jnp.max(t, axis=1, keepdims=True)
    lse = m + jnp.log(jnp.sum(jnp.exp(t - m), axis=1, keepdims=True))
    out_ref[...] = t - lse


def _segsum(p, src, dst, zeros):
    return _make_segsum()(p, src, dst, zeros)


def kernel(x, edge_index, W1_l, W1_r, b1, W2_l, W2_r, b2):
    src = edge_index[0]
    dst = edge_index[1]

    b1r = b1.reshape(1, D_HID)
    b2r = b2.reshape(1, D_OUT)
    zeros = jnp.zeros((NPAD, 128), jnp.float32)

    p1 = pl.pallas_call(
        _pre_body,
        grid=(_NBLK,),
        in_specs=[pl.BlockSpec((_BR, D_IN), lambda i: (i, 0)),
                  pl.BlockSpec((D_IN, D_HID), lambda i: (0, 0))],
        out_specs=pl.BlockSpec((_BR, D_HID), lambda i: (i, 0)),
        out_shape=jax.ShapeDtypeStruct((N, D_HID), jnp.float32),
    )(x, W1_l)

    cntg = pl.pallas_call(
        _hist_body,
        grid=(_NHBLK,),
        in_specs=[pl.BlockSpec((_HCH, 1), lambda i: (i, 0))],
        out_specs=pl.BlockSpec((128, 128), lambda i: (0, 0)),
        out_shape=jax.ShapeDtypeStruct((128, 128), jnp.float32),
    )(dst.reshape(E, 1))
    cnt4 = cntg.reshape(128, 1, 128)

    s1p = _segsum(p1, src, dst, zeros)

    h, p2 = pl.pallas_call(
        _mid_body,
        grid=(_NBLK,),
        in_specs=[pl.BlockSpec((_NSC, _BR, 128), lambda i: (0, i, 0)),
                  pl.BlockSpec((1, 1, 128), lambda i: (i, 0, 0)),
                  pl.BlockSpec((_BR, D_IN), lambda i: (i, 0)),
                  pl.BlockSpec((D_IN, D_HID), lambda i: (0, 0)),
                  pl.BlockSpec((1, D_HID), lambda i: (0, 0)),
                  pl.BlockSpec((D_HID, D_OUT), lambda i: (0, 0))],
        out_specs=[pl.BlockSpec((_BR, D_HID), lambda i: (i, 0)),
                   pl.BlockSpec((_BR, 128), lambda i: (i, 0))],
        out_shape=[jax.ShapeDtypeStruct((N, D_HID), jnp.float32),
                   jax.ShapeDtypeStruct((N, 128), jnp.float32)],
    )(s1p, cnt4, x, W1_r, b1r, W2_l)

    s2p = _segsum(p2, src, dst, zeros)

    out = pl.pallas_call(
        _out_body,
        grid=(_NBLK,),
        in_specs=[pl.BlockSpec((_NSC, _BR, 128), lambda i: (0, i, 0)),
                  pl.BlockSpec((1, 1, 128), lambda i: (i, 0, 0)),
                  pl.BlockSpec((_BR, D_HID), lambda i: (i, 0)),
                  pl.BlockSpec((D_HID, D_OUT), lambda i: (0, 0)),
                  pl.BlockSpec((1, D_OUT), lambda i: (0, 0))],
        out_specs=pl.BlockSpec((_BR, D_OUT), lambda i: (i, 0)),
        out_shape=jax.ShapeDtypeStruct((N, D_OUT), jnp.float32),
    )(s2p, cnt4, h, W2_r, b2r)
    return out


# speedup vs baseline: 1.1503x; 1.0864x over previous
import functools

import jax
import jax.numpy as jnp
from jax import lax
from jax.experimental import pallas as pl
from jax.experimental.pallas import tpu as pltpu
from jax.experimental.pallas import tpu_sc as plsc

N = 10000
E = 320000
D_IN = 128
D_HID = 128
D_OUT = 64
NPAD = 10240
NROW = 80

_NSC = 2
_NTILE = 16
_EDGES_PER_SC = E // _NSC
_EDGES_PER_TILE = _EDGES_PER_SC // _NTILE
_K = 80
_NCHUNK = _EDGES_PER_TILE // _K
_ROWS_PER_TILE = NPAD // _NTILE

_BR = 128
_NBLK = (N + _BR - 1) // _BR


@functools.lru_cache(maxsize=None)
def _make_segsum():
    mesh = plsc.VectorSubcoreMesh(core_axis_name="c", subcore_axis_name="s")

    @functools.partial(
        pl.kernel,
        out_type=jax.ShapeDtypeStruct((_NSC, NPAD, 128), jnp.float32),
        mesh=mesh,
        scratch_types=[
            pltpu.VMEM((_K,), jnp.int32),
            pltpu.VMEM((_K,), jnp.int32),
            pltpu.VMEM((_K, 128), jnp.float32),
            pltpu.VMEM((_K,), jnp.int32),
            pltpu.VMEM((_K,), jnp.int32),
            pltpu.VMEM((_K, 128), jnp.float32),
            pltpu.VMEM((_K,), jnp.int32),
            pltpu.VMEM((_K,), jnp.int32),
            pltpu.VMEM((_K, 128), jnp.float32),
            pltpu.VMEM_SHARED((NPAD, 128), jnp.float32),
        ] + [pltpu.SemaphoreType.DMA] * 6,
    )
    def seg(p_hbm, src_hbm, dst_hbm, zero_hbm, out_hbm,
            src_a, dst_a, rows_a, src_b, dst_b, rows_b,
            src_c, dst_c, rows_c, acc_sh,
            sem_ia, sem_ib, sem_ic, sem_ga, sem_gb, sem_gc):
        c = lax.axis_index("c")
        s = lax.axis_index("s")
        r0 = s * _ROWS_PER_TILE

        pltpu.sync_copy(zero_hbm.at[pl.ds(r0, _ROWS_PER_TILE)],
                        acc_sh.at[pl.ds(r0, _ROWS_PER_TILE)])

        base0 = c * _EDGES_PER_SC + s * _EDGES_PER_TILE
        srcs = (src_a, src_b, src_c)
        dsts = (dst_a, dst_b, dst_c)
        rows = (rows_a, rows_b, rows_c)
        sem_i = (sem_ia, sem_ib, sem_ic)
        sem_g = (sem_ga, sem_gb, sem_gc)

        def idx_fire(i, k):
            b = base0 + i * _K
            pltpu.async_copy(src_hbm.at[pl.ds(b, _K)], srcs[k], sem_i[k])
            pltpu.async_copy(dst_hbm.at[pl.ds(b, _K)], dsts[k], sem_i[k])

        def idx_wait(i, k):
            b = base0 + i * _K
            pltpu.make_async_copy(src_hbm.at[pl.ds(b, _K)], srcs[k],
                                  sem_i[k]).wait()
            pltpu.make_async_copy(dst_hbm.at[pl.ds(b, _K)], dsts[k],
                                  sem_i[k]).wait()

        def gather_fire(k):
            pltpu.async_copy(p_hbm.at[srcs[k]], rows[k], sem_g[k])

        def gather_wait(k):
            pltpu.make_async_copy(p_hbm.at[srcs[k]], rows[k],
                                  sem_g[k]).wait()

        def scatter(k):
            pltpu.sync_copy(rows[k], acc_sh.at[dsts[k]], add=True)

        idx_fire(0, 0)
        idx_fire(1, 1)
        idx_wait(0, 0)
        gather_fire(0)
        idx_wait(1, 1)
        gather_fire(1)
        idx_fire(2, 2)
        plsc.subcore_barrier()

        def body(t, carry):
            for u in range(3):
                g = 3 * t + u
                this, prev = u, (u + 2) % 3
                idx_wait(g + 2, prev)
                gather_fire(prev)
                gather_wait(this)
                scatter(this)

                @pl.when(g + 3 < _NCHUNK)
                def _(g=g, this=this):
                    idx_fire(g + 3, this)
            return carry

        lax.fori_loop(0, (_NCHUNK - 2) // 3, body, 0)
        gather_wait(0)
        scatter(0)
        gather_wait(1)
        scatter(1)
        plsc.subcore_barrier()
        pltpu.sync_copy(acc_sh.at[pl.ds(r0, _ROWS_PER_TILE)],
                        out_hbm.at[c, pl.ds(r0, _ROWS_PER_TILE)])

    return seg


_HCH = 4000
_NHBLK = E // _HCH


def _hist_body(d_ref, out_ref):
    i = pl.program_id(0)
    d = d_ref[...]
    lane = lax.broadcasted_iota(jnp.int32, (1, 128), 1)
    ohh = (lax.shift_right_logical(d, 7) == lane).astype(jnp.bfloat16)
    ohl = (lax.bitwise_and(d, 127) == lane).astype(jnp.bfloat16)
    g = lax.dot_general(ohh, ohl, (((0,), (0,)), ((), ())),
                        preferred_element_type=jnp.float32)

    @pl.when(i == 0)
    def _():
        out_ref[...] = g

    @pl.when(i > 0)
    def _():
        out_ref[...] += g


def _pre_body(x_ref, wl_ref, out_ref):
    out_ref[...] = jnp.dot(x_ref[...], wl_ref[...],
                           preferred_element_type=jnp.float32)


def _inv_diag(cp_ref):
    cnt = cp_ref[0, 0]
    inv = 1.0 / jnp.maximum(cnt, 1.0)
    return jnp.eye(_BR, dtype=jnp.float32) * inv[None, :]


def _mid_body(sp_ref, cp_ref, x_ref, wr_ref, b1_ref, w2l_ref, h_ref, p2_ref):
    ssum = sp_ref[0] + sp_ref[1]
    mean_w = jnp.dot(_inv_diag(cp_ref), ssum,
                     preferred_element_type=jnp.float32)
    h = mean_w + jnp.dot(x_ref[...], wr_ref[...],
                         preferred_element_type=jnp.float32) + b1_ref[...]
    h = jnp.maximum(h, 0.0)
    h_ref[...] = h
    p2 = jnp.dot(h, w2l_ref[...], preferred_element_type=jnp.float32)
    p2_ref[...] = jnp.concatenate(
        [p2, jnp.zeros((_BR, 128 - D_OUT), jnp.float32)], axis=1)


def _out_body(sp_ref, cp_ref, h_ref, w2r_ref, b2_ref, out_ref):
    ssum = sp_ref[0, :, :D_OUT] + sp_ref[1, :, :D_OUT]
    t = (jnp.dot(_inv_diag(cp_ref), ssum, preferred_element_type=jnp.float32)
         + jnp.dot(h_ref[...], w2r_ref[...], preferred_element_type=jnp.float32)
         + b2_ref[...])
    m = jnp.max(t, axis=1, keepdims=True)
    lse = m + jnp.log(jnp.sum(jnp.exp(t - m), axis=1, keepdims=True))
    out_ref[...] = t - lse


def _segsum(p, src, dst, zeros):
    return _make_segsum()(p, src, dst, zeros)


def kernel(x, edge_index, W1_l, W1_r, b1, W2_l, W2_r, b2):
    src = edge_index[0]
    dst = edge_index[1]

    b1r = b1.reshape(1, D_HID)
    b2r = b2.reshape(1, D_OUT)
    zeros = jnp.zeros((NPAD, 128), jnp.float32)

    p1 = pl.pallas_call(
        _pre_body,
        grid=(_NBLK,),
        in_specs=[pl.BlockSpec((_BR, D_IN), lambda i: (i, 0)),
                  pl.BlockSpec((D_IN, D_HID), lambda i: (0, 0))],
        out_specs=pl.BlockSpec((_BR, D_HID), lambda i: (i, 0)),
        out_shape=jax.ShapeDtypeStruct((N, D_HID), jnp.float32),
    )(x, W1_l)

    s1p = _segsum(p1, src, dst, zeros)

    cntg = pl.pallas_call(
        _hist_body,
        grid=(_NHBLK,),
        in_specs=[pl.BlockSpec((_HCH, 1), lambda i: (i, 0))],
        out_specs=pl.BlockSpec((128, 128), lambda i: (0, 0)),
        out_shape=jax.ShapeDtypeStruct((128, 128), jnp.float32),
    )(dst.reshape(E, 1))
    cnt4 = cntg.reshape(128, 1, 128)


    h, p2 = pl.pallas_call(
        _mid_body,
        grid=(_NBLK,),
        in_specs=[pl.BlockSpec((_NSC, _BR, 128), lambda i: (0, i, 0)),
                  pl.BlockSpec((1, 1, 128), lambda i: (i, 0, 0)),
                  pl.BlockSpec((_BR, D_IN), lambda i: (i, 0)),
                  pl.BlockSpec((D_IN, D_HID), lambda i: (0, 0)),
                  pl.BlockSpec((1, D_HID), lambda i: (0, 0)),
                  pl.BlockSpec((D_HID, D_OUT), lambda i: (0, 0))],
        out_specs=[pl.BlockSpec((_BR, D_HID), lambda i: (i, 0)),
                   pl.BlockSpec((_BR, 128), lambda i: (i, 0))],
        out_shape=[jax.ShapeDtypeStruct((N, D_HID), jnp.float32),
                   jax.ShapeDtypeStruct((N, 128), jnp.float32)],
    )(s1p, cnt4, x, W1_r, b1r, W2_l)

    s2p = _segsum(p2, src, dst, zeros)

    out = pl.pallas_call(
        _out_body,
        grid=(_NBLK,),
        in_specs=[pl.BlockSpec((_NSC, _BR, 128), lambda i: (0, i, 0)),
                  pl.BlockSpec((1, 1, 128), lambda i: (i, 0, 0)),
                  pl.BlockSpec((_BR, D_HID), lambda i: (i, 0)),
                  pl.BlockSpec((D_HID, D_OUT), lambda i: (0, 0)),
                  pl.BlockSpec((1, D_OUT), lambda i: (0, 0))],
        out_specs=pl.BlockSpec((_BR, D_OUT), lambda i: (i, 0)),
        out_shape=jax.ShapeDtypeStruct((N, D_OUT), jnp.float32),
    )(s2p, cnt4, h, W2_r, b2r)
    return out


# speedup vs baseline: 1.1805x; 1.0263x over previous
import functools

import jax
import jax.numpy as jnp
from jax import lax
from jax.experimental import pallas as pl
from jax.experimental.pallas import tpu as pltpu
from jax.experimental.pallas import tpu_sc as plsc

N = 10000
E = 320000
D_IN = 128
D_HID = 128
D_OUT = 64
NPAD = 10240
NROW = 80

_NSC = 2
_NTILE = 16
_EDGES_PER_SC = E // _NSC
_EDGES_PER_TILE = _EDGES_PER_SC // _NTILE
_K = 80
_NCHUNK = _EDGES_PER_TILE // _K
_ROWS_PER_TILE = NPAD // _NTILE

_BR = 128
_NBLK = (N + _BR - 1) // _BR


@functools.lru_cache(maxsize=None)
def _make_segsum():
    mesh = plsc.VectorSubcoreMesh(core_axis_name="c", subcore_axis_name="s")

    @functools.partial(
        pl.kernel,
        out_type=jax.ShapeDtypeStruct((_NSC, NPAD, 128), jnp.float32),
        mesh=mesh,
        scratch_types=[
            pltpu.VMEM((_K,), jnp.int32),
            pltpu.VMEM((_K,), jnp.int32),
            pltpu.VMEM((_K, 128), jnp.float32),
            pltpu.VMEM((_K,), jnp.int32),
            pltpu.VMEM((_K,), jnp.int32),
            pltpu.VMEM((_K, 128), jnp.float32),
            pltpu.VMEM((_K,), jnp.int32),
            pltpu.VMEM((_K,), jnp.int32),
            pltpu.VMEM((_K, 128), jnp.float32),
            pltpu.VMEM_SHARED((NPAD, 128), jnp.float32),
        ] + [pltpu.SemaphoreType.DMA] * 6,
    )
    def seg(p_hbm, src_hbm, dst_hbm, zero_hbm, out_hbm,
            src_a, dst_a, rows_a, src_b, dst_b, rows_b,
            src_c, dst_c, rows_c, acc_sh,
            sem_ia, sem_ib, sem_ic, sem_ga, sem_gb, sem_gc):
        c = lax.axis_index("c")
        s = lax.axis_index("s")
        r0 = s * _ROWS_PER_TILE

        pltpu.sync_copy(zero_hbm.at[pl.ds(r0, _ROWS_PER_TILE)],
                        acc_sh.at[pl.ds(r0, _ROWS_PER_TILE)])

        base0 = c * _EDGES_PER_SC + s * _EDGES_PER_TILE
        srcs = (src_a, src_b, src_c)
        dsts = (dst_a, dst_b, dst_c)
        rows = (rows_a, rows_b, rows_c)
        sem_i = (sem_ia, sem_ib, sem_ic)
        sem_g = (sem_ga, sem_gb, sem_gc)

        def idx_fire(i, k):
            b = base0 + i * _K
            pltpu.async_copy(src_hbm.at[pl.ds(b, _K)], srcs[k], sem_i[k])
            pltpu.async_copy(dst_hbm.at[pl.ds(b, _K)], dsts[k], sem_i[k])

        def idx_wait(i, k):
            b = base0 + i * _K
            pltpu.make_async_copy(src_hbm.at[pl.ds(b, _K)], srcs[k],
                                  sem_i[k]).wait()
            pltpu.make_async_copy(dst_hbm.at[pl.ds(b, _K)], dsts[k],
                                  sem_i[k]).wait()

        def gather_fire(k):
            pltpu.async_copy(p_hbm.at[srcs[k]], rows[k], sem_g[k])

        def gather_wait(k):
            pltpu.make_async_copy(p_hbm.at[srcs[k]], rows[k],
                                  sem_g[k]).wait()

        def scatter(k):
            pltpu.sync_copy(rows[k], acc_sh.at[dsts[k]], add=True)

        idx_fire(0, 0)
        idx_fire(1, 1)
        idx_wait(0, 0)
        gather_fire(0)
        idx_wait(1, 1)
        gather_fire(1)
        idx_fire(2, 2)
        plsc.subcore_barrier()

        def body(t, carry):
            for u in range(3):
                g = 3 * t + u
                this, prev = u, (u + 2) % 3
                idx_wait(g + 2, prev)
                gather_fire(prev)
                gather_wait(this)
                scatter(this)

                @pl.when(g + 3 < _NCHUNK)
                def _(g=g, this=this):
                    idx_fire(g + 3, this)
            return carry

        lax.fori_loop(0, (_NCHUNK - 2) // 3, body, 0)
        gather_wait(0)
        scatter(0)
        gather_wait(1)
        scatter(1)
        plsc.subcore_barrier()
        pltpu.sync_copy(acc_sh.at[pl.ds(r0, _ROWS_PER_TILE)],
                        out_hbm.at[c, pl.ds(r0, _ROWS_PER_TILE)])

    return seg


_HCH = 8000
_NHBLK = E // _HCH


def _hist_body(d_ref, out_ref):
    i = pl.program_id(0)
    d = d_ref[...]
    lane = lax.broadcasted_iota(jnp.int32, (1, 128), 1)
    ohh = (lax.shift_right_logical(d, 7) == lane).astype(jnp.bfloat16)
    ohl = (lax.bitwise_and(d, 127) == lane).astype(jnp.bfloat16)
    g = lax.dot_general(ohh, ohl, (((0,), (0,)), ((), ())),
                        preferred_element_type=jnp.float32)

    @pl.when(i == 0)
    def _():
        out_ref[...] = g

    @pl.when(i > 0)
    def _():
        out_ref[...] += g


def _pre_body(x_ref, wl_ref, out_ref):
    out_ref[...] = jnp.dot(x_ref[...], wl_ref[...],
                           preferred_element_type=jnp.float32)


def _inv_diag(cp_ref):
    cnt = cp_ref[0, 0]
    inv = 1.0 / jnp.maximum(cnt, 1.0)
    return jnp.eye(_BR, dtype=jnp.float32) * inv[None, :]


def _mid_body(sp_ref, cp_ref, x_ref, wr_ref, b1_ref, w2l_ref, h_ref, p2_ref):
    ssum = sp_ref[0] + sp_ref[1]
    mean_w = jnp.dot(_inv_diag(cp_ref), ssum,
                     preferred_element_type=jnp.float32)
    h = mean_w + jnp.dot(x_ref[...], wr_ref[...],
                         preferred_element_type=jnp.float32) + b1_ref[...]
    h = jnp.maximum(h, 0.0)
    h_ref[...] = h
    p2 = jnp.dot(h, w2l_ref[...], preferred_element_type=jnp.float32)
    p2_ref[...] = jnp.concatenate(
        [p2, jnp.zeros((_BR, 128 - D_OUT), jnp.float32)], axis=1)


def _out_body(sp_ref, cp_ref, h_ref, w2r_ref, b2_ref, out_ref):
    ssum = sp_ref[0, :, :D_OUT] + sp_ref[1, :, :D_OUT]
    t = (jnp.dot(_inv_diag(cp_ref), ssum, preferred_element_type=jnp.float32)
         + jnp.dot(h_ref[...], w2r_ref[...], preferred_element_type=jnp.float32)
         + b2_ref[...])
    m = jnp.max(t, axis=1, keepdims=True)
    lse = m + jnp.log(jnp.sum(jnp.exp(t - m), axis=1, keepdims=True))
    out_ref[...] = t - lse


def _segsum(p, src, dst, zeros):
    return _make_segsum()(p, src, dst, zeros)


def kernel(x, edge_index, W1_l, W1_r, b1, W2_l, W2_r, b2):
    src = edge_index[0]
    dst = edge_index[1]

    b1r = b1.reshape(1, D_HID)
    b2r = b2.reshape(1, D_OUT)
    zeros = jnp.zeros((NPAD, 128), jnp.float32)

    p1 = pl.pallas_call(
        _pre_body,
        grid=(_NBLK,),
        in_specs=[pl.BlockSpec((_BR, D_IN), lambda i: (i, 0)),
                  pl.BlockSpec((D_IN, D_HID), lambda i: (0, 0))],
        out_specs=pl.BlockSpec((_BR, D_HID), lambda i: (i, 0)),
        out_shape=jax.ShapeDtypeStruct((N, D_HID), jnp.float32),
    )(x, W1_l)

    s1p = _segsum(p1, src, dst, zeros)

    cntg = pl.pallas_call(
        _hist_body,
        grid=(_NHBLK,),
        in_specs=[pl.BlockSpec((_HCH, 1), lambda i: (i, 0))],
        out_specs=pl.BlockSpec((128, 128), lambda i: (0, 0)),
        out_shape=jax.ShapeDtypeStruct((128, 128), jnp.float32),
    )(dst.reshape(E, 1))
    cnt4 = cntg.reshape(128, 1, 128)


    h, p2 = pl.pallas_call(
        _mid_body,
        grid=(_NBLK,),
        in_specs=[pl.BlockSpec((_NSC, _BR, 128), lambda i: (0, i, 0)),
                  pl.BlockSpec((1, 1, 128), lambda i: (i, 0, 0)),
                  pl.BlockSpec((_BR, D_IN), lambda i: (i, 0)),
                  pl.BlockSpec((D_IN, D_HID), lambda i: (0, 0)),
                  pl.BlockSpec((1, D_HID), lambda i: (0, 0)),
                  pl.BlockSpec((D_HID, D_OUT), lambda i: (0, 0))],
        out_specs=[pl.BlockSpec((_BR, D_HID), lambda i: (i, 0)),
                   pl.BlockSpec((_BR, 128), lambda i: (i, 0))],
        out_shape=[jax.ShapeDtypeStruct((N, D_HID), jnp.float32),
                   jax.ShapeDtypeStruct((N, 128), jnp.float32)],
    )(s1p, cnt4, x, W1_r, b1r, W2_l)

    s2p = _segsum(p2, src, dst, zeros)

    out = pl.pallas_call(
        _out_body,
        grid=(_NBLK,),
        in_specs=[pl.BlockSpec((_NSC, _BR, 128), lambda i: (0, i, 0)),
                  pl.BlockSpec((1, 1, 128), lambda i: (i, 0, 0)),
                  pl.BlockSpec((_BR, D_HID), lambda i: (i, 0)),
                  pl.BlockSpec((D_HID, D_OUT), lambda i: (0, 0)),
                  pl.BlockSpec((1, D_OUT), lambda i: (0, 0))],
        out_specs=pl.BlockSpec((_BR, D_OUT), lambda i: (i, 0)),
        out_shape=jax.ShapeDtypeStruct((N, D_OUT), jnp.float32),
    )(s2p, cnt4, h, W2_r, b2r)
    return out
